# baseline (speedup 1.0000x reference)
import functools

import jax
import jax.numpy as jnp
from jax import lax
from jax.experimental import pallas as pl
from jax.experimental.pallas import tpu as pltpu
from jax.experimental.pallas import tpu_sc as plsc

VOCAB = 1000000
EMBED_DIM = 64
PAD_DIM = 128
BATCH = 16384
HIST = 50

_NW = 32
_B_PER_W = BATCH // _NW
_BCH = 8
_CHUNK = _BCH * HIST
_NCH = _B_PER_W // _BCH
_NPAIR = _NCH // 2


_TSTEP = 2048
_TGRID = -(-VOCAB // _TSTEP)


def _padtrans_body(in_ref, out_ref):
    eye = jax.lax.broadcasted_iota(jnp.int32, (EMBED_DIM, EMBED_DIM), 0)
    eye = (eye == jax.lax.broadcasted_iota(
        jnp.int32, (EMBED_DIM, EMBED_DIM), 1)).astype(jnp.float32)
    t = jax.lax.dot_general(in_ref[...], eye, (((0,), (0,)), ((), ())),
                            precision=jax.lax.Precision.HIGHEST,
                            preferred_element_type=jnp.float32)
    out_ref[:, : EMBED_DIM] = t


_padtrans = pl.pallas_call(
    _padtrans_body,
    grid=(_TGRID,),
    in_specs=[pl.BlockSpec((EMBED_DIM, _TSTEP), lambda i: (0, i))],
    out_specs=pl.BlockSpec((_TSTEP, PAD_DIM), lambda i: (i, 0)),
    out_shape=jax.ShapeDtypeStruct((VOCAB, PAD_DIM), jnp.float32),
)


def _make_gather():
    mesh = plsc.VectorSubcoreMesh(core_axis_name="c", subcore_axis_name="s")

    @functools.partial(
        pl.kernel,
        out_type=jax.ShapeDtypeStruct((BATCH, HIST, EMBED_DIM), jnp.float32),
        scratch_types=[
            pltpu.VMEM((_CHUNK,), jnp.int32),
            pltpu.VMEM((_CHUNK,), jnp.int32),
            pltpu.VMEM((_CHUNK, PAD_DIM), jnp.float32),
            pltpu.VMEM((_CHUNK, PAD_DIM), jnp.float32),
            pltpu.SemaphoreType.DMA,
            pltpu.SemaphoreType.DMA,
            pltpu.SemaphoreType.DMA,
            pltpu.SemaphoreType.DMA,
        ],
        mesh=mesh,
        compiler_params=pltpu.CompilerParams(use_tc_tiling_on_sc=False),
    )
    def gather_kernel(idx_hbm, table_hbm, out_hbm,
                      idx0, idx1, rows0, rows1, g0, g1, s0, s1):
        wid = lax.axis_index("s") * 2 + lax.axis_index("c")
        base_b = wid * _B_PER_W
        idx_v = (idx0, idx1)
        rows_v = (rows0, rows1)
        gsem = (g0, g1)
        ssem = (s0, s1)

        def issue_gather(b, i):
            off = (base_b + i * _BCH) * HIST
            pltpu.sync_copy(idx_hbm.at[pl.ds(off, _CHUNK)], idx_v[b])
            pltpu.async_copy(table_hbm.at[idx_v[b]], rows_v[b], gsem[b])

        def wait_gather(b):
            pltpu.make_async_copy(table_hbm.at[idx_v[b]], rows_v[b],
                                  gsem[b]).wait()

        def issue_store(b, i):
            b0 = base_b + i * _BCH
            for k in range(_BCH):
                pltpu.async_copy(
                    rows_v[b].at[pl.ds(k * HIST, HIST), pl.ds(0, EMBED_DIM)],
                    out_hbm.at[b0 + k], ssem[b])

        def wait_store(b, i):
            b0 = base_b + i * _BCH
            for k in range(_BCH):
                pltpu.make_async_copy(
                    rows_v[b].at[pl.ds(k * HIST, HIST), pl.ds(0, EMBED_DIM)],
                    out_hbm.at[b0 + k], ssem[b]).wait()

        issue_gather(0, 0)

        def body(j, carry):
            i0 = 2 * j
            i1 = i0 + 1
            wait_gather(0)
            issue_store(0, i0)

            @pl.when(j > 0)
            def _():
                wait_store(1, i0 - 1)

            issue_gather(1, i1)
            wait_gather(1)
            issue_store(1, i1)
            wait_store(0, i0)

            @pl.when(j < _NPAIR - 1)
            def _():
                issue_gather(0, i0 + 2)

            return carry

        lax.fori_loop(0, _NPAIR, body, 0)
        wait_store(1, _NCH - 1)

    return gather_kernel


_gather = _make_gather()


def kernel(x, table):
    idx = x.reshape(-1).astype(jnp.int32)
    tbl = _padtrans(table.T)
    return _gather(idx, tbl)


# speedup vs baseline: 1.3577x; 1.3577x over previous
import functools

import jax
import jax.numpy as jnp
from jax import lax
from jax.experimental import pallas as pl
from jax.experimental.pallas import tpu as pltpu
from jax.experimental.pallas import tpu_sc as plsc

VOCAB = 1000000
EMBED_DIM = 64
PAD_DIM = 128
BATCH = 16384
HIST = 50

_NW = 32
_B_PER_W = BATCH // _NW
_BCH = 8
_CHUNK = _BCH * HIST
_NCH = _B_PER_W // _BCH
_NPAIR = _NCH // 2


_TSTEP = 8192
_TGRID = -(-VOCAB // _TSTEP)


def _detranspose_body(in_ref, out_ref):
    t = in_ref[...].T
    t3 = t.reshape(_TSTEP // 2, 2, EMBED_DIM)
    out_ref[...] = jnp.concatenate([t3[:, 0, :], t3[:, 1, :]], axis=-1)


_detranspose = pl.pallas_call(
    _detranspose_body,
    grid=(_TGRID,),
    in_specs=[pl.BlockSpec((EMBED_DIM, _TSTEP), lambda i: (0, i))],
    out_specs=pl.BlockSpec((_TSTEP // 2, 128), lambda i: (i, 0)),
    out_shape=jax.ShapeDtypeStruct((VOCAB // 2, 128), jnp.float32),
)


def _make_gather():
    mesh = plsc.VectorSubcoreMesh(core_axis_name="c", subcore_axis_name="s")

    @functools.partial(
        pl.kernel,
        out_type=jax.ShapeDtypeStruct((BATCH, HIST, EMBED_DIM), jnp.float32),
        scratch_types=[
            pltpu.VMEM((_CHUNK,), jnp.int32),
            pltpu.VMEM((_CHUNK,), jnp.int32),
            pltpu.VMEM((_CHUNK, EMBED_DIM), jnp.float32),
            pltpu.VMEM((_CHUNK, EMBED_DIM), jnp.float32),
            pltpu.SemaphoreType.DMA,
            pltpu.SemaphoreType.DMA,
            pltpu.SemaphoreType.DMA,
            pltpu.SemaphoreType.DMA,
        ],
        mesh=mesh,
        compiler_params=pltpu.CompilerParams(use_tc_tiling_on_sc=False),
    )
    def gather_kernel(idx_hbm, table_hbm, out_hbm,
                      idx0, idx1, rows0, rows1, g0, g1, s0, s1):
        wid = lax.axis_index("s") * 2 + lax.axis_index("c")
        base_b = wid * _B_PER_W
        idx_v = (idx0, idx1)
        rows_v = (rows0, rows1)
        gsem = (g0, g1)
        ssem = (s0, s1)

        def issue_gather(b, i):
            off = (base_b + i * _BCH) * HIST
            pltpu.sync_copy(idx_hbm.at[pl.ds(off, _CHUNK)], idx_v[b])
            pltpu.async_copy(table_hbm.at[idx_v[b]], rows_v[b], gsem[b])

        def wait_gather(b):
            pltpu.make_async_copy(table_hbm.at[idx_v[b]], rows_v[b],
                                  gsem[b]).wait()

        def issue_store(b, i):
            b0 = base_b + i * _BCH
            for k in range(_BCH):
                pltpu.async_copy(
                    rows_v[b].at[pl.ds(k * HIST, HIST), :],
                    out_hbm.at[b0 + k], ssem[b])

        def wait_store(b, i):
            b0 = base_b + i * _BCH
            for k in range(_BCH):
                pltpu.make_async_copy(
                    rows_v[b].at[pl.ds(k * HIST, HIST), :],
                    out_hbm.at[b0 + k], ssem[b]).wait()

        issue_gather(0, 0)

        def body(j, carry):
            i0 = 2 * j
            i1 = i0 + 1
            wait_gather(0)
            issue_store(0, i0)

            @pl.when(j > 0)
            def _():
                wait_store(1, i0 - 1)

            issue_gather(1, i1)
            wait_gather(1)
            issue_store(1, i1)
            wait_store(0, i0)

            @pl.when(j < _NPAIR - 1)
            def _():
                issue_gather(0, i0 + 2)

            return carry

        lax.fori_loop(0, _NPAIR, body, 0)
        wait_store(1, _NCH - 1)

    return gather_kernel


_gather = _make_gather()


def kernel(x, table):
    idx = x.reshape(-1).astype(jnp.int32)
    tbl = _detranspose(table.T).reshape(VOCAB, EMBED_DIM)
    return _gather(idx, tbl)


# speedup vs baseline: 1.3592x; 1.0011x over previous
import functools

import jax
import jax.numpy as jnp
from jax import lax
from jax.experimental import pallas as pl
from jax.experimental.pallas import tpu as pltpu
from jax.experimental.pallas import tpu_sc as plsc

VOCAB = 1000000
EMBED_DIM = 64
PAD_DIM = 128
BATCH = 16384
HIST = 50

_NW = 32
_B_PER_W = BATCH // _NW
_BCH = 8
_CHUNK = _BCH * HIST
_NCH = _B_PER_W // _BCH
_NPAIR = _NCH // 2


_TSTEP = 16384
_TGRID = -(-VOCAB // _TSTEP)


def _detranspose_body(in_ref, out_ref):
    t = in_ref[...].T
    t3 = t.reshape(_TSTEP // 2, 2, EMBED_DIM)
    out_ref[...] = jnp.concatenate([t3[:, 0, :], t3[:, 1, :]], axis=-1)


_detranspose = pl.pallas_call(
    _detranspose_body,
    grid=(_TGRID,),
    in_specs=[pl.BlockSpec((EMBED_DIM, _TSTEP), lambda i: (0, i))],
    out_specs=pl.BlockSpec((_TSTEP // 2, 128), lambda i: (i, 0)),
    out_shape=jax.ShapeDtypeStruct((VOCAB // 2, 128), jnp.float32),
)


def _make_gather():
    mesh = plsc.VectorSubcoreMesh(core_axis_name="c", subcore_axis_name="s")

    @functools.partial(
        pl.kernel,
        out_type=jax.ShapeDtypeStruct((BATCH, HIST, EMBED_DIM), jnp.float32),
        scratch_types=[
            pltpu.VMEM((_CHUNK,), jnp.int32),
            pltpu.VMEM((_CHUNK,), jnp.int32),
            pltpu.VMEM((_CHUNK, EMBED_DIM), jnp.float32),
            pltpu.VMEM((_CHUNK, EMBED_DIM), jnp.float32),
            pltpu.SemaphoreType.DMA,
            pltpu.SemaphoreType.DMA,
            pltpu.SemaphoreType.DMA,
            pltpu.SemaphoreType.DMA,
        ],
        mesh=mesh,
        compiler_params=pltpu.CompilerParams(use_tc_tiling_on_sc=False),
    )
    def gather_kernel(idx_hbm, table_hbm, out_hbm,
                      idx0, idx1, rows0, rows1, g0, g1, s0, s1):
        wid = lax.axis_index("s") * 2 + lax.axis_index("c")
        base_b = wid * _B_PER_W
        idx_v = (idx0, idx1)
        rows_v = (rows0, rows1)
        gsem = (g0, g1)
        ssem = (s0, s1)

        def issue_gather(b, i):
            off = (base_b + i * _BCH) * HIST
            pltpu.sync_copy(idx_hbm.at[pl.ds(off, _CHUNK)], idx_v[b])
            pltpu.async_copy(table_hbm.at[idx_v[b]], rows_v[b], gsem[b])

        def wait_gather(b):
            pltpu.make_async_copy(table_hbm.at[idx_v[b]], rows_v[b],
                                  gsem[b]).wait()

        def issue_store(b, i):
            b0 = base_b + i * _BCH
            for k in range(_BCH):
                pltpu.async_copy(
                    rows_v[b].at[pl.ds(k * HIST, HIST), :],
                    out_hbm.at[b0 + k], ssem[b])

        def wait_store(b, i):
            b0 = base_b + i * _BCH
            for k in range(_BCH):
                pltpu.make_async_copy(
                    rows_v[b].at[pl.ds(k * HIST, HIST), :],
                    out_hbm.at[b0 + k], ssem[b]).wait()

        issue_gather(0, 0)

        def body(j, carry):
            i0 = 2 * j
            i1 = i0 + 1
            wait_gather(0)
            issue_store(0, i0)

            @pl.when(j > 0)
            def _():
                wait_store(1, i0 - 1)

            issue_gather(1, i1)
            wait_gather(1)
            issue_store(1, i1)
            wait_store(0, i0)

            @pl.when(j < _NPAIR - 1)
            def _():
                issue_gather(0, i0 + 2)

            return carry

        lax.fori_loop(0, _NPAIR, body, 0)
        wait_store(1, _NCH - 1)

    return gather_kernel


_gather = _make_gather()


def kernel(x, table):
    idx = x.reshape(-1).astype(jnp.int32)
    tbl = _detranspose(table.T).reshape(VOCAB, EMBED_DIM)
    return _gather(idx, tbl)


# speedup vs baseline: 1.3598x; 1.0004x over previous
import functools

import jax
import jax.numpy as jnp
from jax import lax
from jax.experimental import pallas as pl
from jax.experimental.pallas import tpu as pltpu
from jax.experimental.pallas import tpu_sc as plsc

VOCAB = 1000000
EMBED_DIM = 64
BATCH = 16384
HIST = 50

_NW = 32
_B_PER_W = BATCH // _NW
_BCH = 8
_CHUNK = _BCH * HIST
_NCH = _B_PER_W // _BCH
_NPAIR = _NCH // 2


_TSTEP = 16384
_TGRID = -(-VOCAB // _TSTEP)


def _detranspose_body(in_ref, out_ref):
    t = in_ref[...].T
    t3 = t.reshape(_TSTEP // 2, 2, EMBED_DIM)
    out_ref[...] = jnp.concatenate([t3[:, 0, :], t3[:, 1, :]], axis=-1)


_detranspose = pl.pallas_call(
    _detranspose_body,
    grid=(_TGRID,),
    in_specs=[pl.BlockSpec((EMBED_DIM, _TSTEP), lambda i: (0, i))],
    out_specs=pl.BlockSpec((_TSTEP // 2, 128), lambda i: (i, 0)),
    out_shape=jax.ShapeDtypeStruct((VOCAB // 2, 128), jnp.float32),
)


def _make_gather():
    mesh = plsc.VectorSubcoreMesh(core_axis_name="c", subcore_axis_name="s")

    @functools.partial(
        pl.kernel,
        out_type=jax.ShapeDtypeStruct((BATCH, HIST, EMBED_DIM), jnp.float32),
        scratch_types=[
            pltpu.VMEM((_CHUNK,), jnp.int32),
            pltpu.VMEM((_CHUNK,), jnp.int32),
            pltpu.VMEM((_CHUNK, EMBED_DIM), jnp.float32),
            pltpu.VMEM((_CHUNK, EMBED_DIM), jnp.float32),
            pltpu.SemaphoreType.DMA,
            pltpu.SemaphoreType.DMA,
            pltpu.SemaphoreType.DMA,
            pltpu.SemaphoreType.DMA,
        ],
        mesh=mesh,
        compiler_params=pltpu.CompilerParams(use_tc_tiling_on_sc=False),
    )
    def gather_kernel(idx_hbm, table_hbm, out_hbm,
                      idx0, idx1, rows0, rows1, g0, g1, s0, s1):
        wid = lax.axis_index("s") * 2 + lax.axis_index("c")
        base_b = wid * _B_PER_W
        idx_v = (idx0, idx1)
        rows_v = (rows0, rows1)
        gsem = (g0, g1)
        ssem = (s0, s1)

        def issue_gather(b, i):
            off = (base_b + i * _BCH) * HIST
            pltpu.sync_copy(idx_hbm.at[pl.ds(off, _CHUNK)], idx_v[b])
            pltpu.async_copy(table_hbm.at[idx_v[b]], rows_v[b], gsem[b])

        def wait_gather(b):
            pltpu.make_async_copy(table_hbm.at[idx_v[b]], rows_v[b],
                                  gsem[b]).wait()

        def issue_store(b, i):
            b0 = base_b + i * _BCH
            for k in range(_BCH):
                pltpu.async_copy(
                    rows_v[b].at[pl.ds(k * HIST, HIST), :],
                    out_hbm.at[b0 + k], ssem[b])

        def wait_store(b, i):
            b0 = base_b + i * _BCH
            for k in range(_BCH):
                pltpu.make_async_copy(
                    rows_v[b].at[pl.ds(k * HIST, HIST), :],
                    out_hbm.at[b0 + k], ssem[b]).wait()

        issue_gather(0, 0)

        def body(j, carry):
            i0 = 2 * j
            i1 = i0 + 1
            wait_gather(0)
            issue_store(0, i0)

            @pl.when(j > 0)
            def _():
                wait_store(1, i0 - 1)

            issue_gather(1, i1)
            wait_gather(1)
            issue_store(1, i1)
            wait_store(0, i0)

            @pl.when(j < _NPAIR - 1)
            def _():
                issue_gather(0, i0 + 2)

            return carry

        lax.fori_loop(0, _NPAIR, body, 0)
        wait_store(1, _NCH - 1)

    return gather_kernel


_gather = _make_gather()


def kernel(x, table):
    idx = x.reshape(-1).astype(jnp.int32)
    tbl = _detranspose(table.T).reshape(VOCAB, EMBED_DIM)
    return _gather(idx, tbl)


# speedup vs baseline: 1.3902x; 1.0224x over previous
import functools

import jax
import jax.numpy as jnp
from jax import lax
from jax.experimental import pallas as pl
from jax.experimental.pallas import tpu as pltpu
from jax.experimental.pallas import tpu_sc as plsc

VOCAB = 1000000
EMBED_DIM = 64
BATCH = 16384
HIST = 50

_NW = 32
_B_PER_W = BATCH // _NW
_BCH = 16
_CHUNK = _BCH * HIST
_NCH = _B_PER_W // _BCH
_NPAIR = _NCH // 2


_TSTEP = 16384
_TGRID = -(-VOCAB // _TSTEP)


def _detranspose_body(in_ref, out_ref):
    t = in_ref[...].T
    t3 = t.reshape(_TSTEP // 2, 2, EMBED_DIM)
    out_ref[...] = jnp.concatenate([t3[:, 0, :], t3[:, 1, :]], axis=-1)


_detranspose = pl.pallas_call(
    _detranspose_body,
    grid=(_TGRID,),
    in_specs=[pl.BlockSpec((EMBED_DIM, _TSTEP), lambda i: (0, i))],
    out_specs=pl.BlockSpec((_TSTEP // 2, 128), lambda i: (i, 0)),
    out_shape=jax.ShapeDtypeStruct((VOCAB // 2, 128), jnp.float32),
)


def _make_gather():
    mesh = plsc.VectorSubcoreMesh(core_axis_name="c", subcore_axis_name="s")

    @functools.partial(
        pl.kernel,
        out_type=jax.ShapeDtypeStruct((BATCH, HIST, EMBED_DIM), jnp.float32),
        scratch_types=[
            pltpu.VMEM((_CHUNK,), jnp.int32),
            pltpu.VMEM((_CHUNK,), jnp.int32),
            pltpu.VMEM((_CHUNK, EMBED_DIM), jnp.float32),
            pltpu.VMEM((_CHUNK, EMBED_DIM), jnp.float32),
            pltpu.SemaphoreType.DMA,
            pltpu.SemaphoreType.DMA,
            pltpu.SemaphoreType.DMA,
            pltpu.SemaphoreType.DMA,
        ],
        mesh=mesh,
        compiler_params=pltpu.CompilerParams(use_tc_tiling_on_sc=False),
    )
    def gather_kernel(idx_hbm, table_hbm, out_hbm,
                      idx0, idx1, rows0, rows1, g0, g1, s0, s1):
        wid = lax.axis_index("s") * 2 + lax.axis_index("c")
        base_b = wid * _B_PER_W
        idx_v = (idx0, idx1)
        rows_v = (rows0, rows1)
        gsem = (g0, g1)
        ssem = (s0, s1)

        def issue_gather(b, i):
            off = (base_b + i * _BCH) * HIST
            pltpu.sync_copy(idx_hbm.at[pl.ds(off, _CHUNK)], idx_v[b])
            pltpu.async_copy(table_hbm.at[idx_v[b]], rows_v[b], gsem[b])

        def wait_gather(b):
            pltpu.make_async_copy(table_hbm.at[idx_v[b]], rows_v[b],
                                  gsem[b]).wait()

        def issue_store(b, i):
            b0 = base_b + i * _BCH
            for k in range(_BCH):
                pltpu.async_copy(
                    rows_v[b].at[pl.ds(k * HIST, HIST), :],
                    out_hbm.at[b0 + k], ssem[b])

        def wait_store(b, i):
            b0 = base_b + i * _BCH
            for k in range(_BCH):
                pltpu.make_async_copy(
                    rows_v[b].at[pl.ds(k * HIST, HIST), :],
                    out_hbm.at[b0 + k], ssem[b]).wait()

        issue_gather(0, 0)

        def body(j, carry):
            i0 = 2 * j
            i1 = i0 + 1
            wait_gather(0)
            issue_store(0, i0)

            @pl.when(j > 0)
            def _():
                wait_store(1, i0 - 1)

            issue_gather(1, i1)
            wait_gather(1)
            issue_store(1, i1)
            wait_store(0, i0)

            @pl.when(j < _NPAIR - 1)
            def _():
                issue_gather(0, i0 + 2)

            return carry

        lax.fori_loop(0, _NPAIR, body, 0)
        wait_store(1, _NCH - 1)

    return gather_kernel


_gather = _make_gather()


def kernel(x, table):
    idx = x.reshape(-1).astype(jnp.int32)
    tbl = _detranspose(table.T).reshape(VOCAB, EMBED_DIM)
    return _gather(idx, tbl)
